# baseline (speedup 1.0000x reference)
import functools
import numpy as np
import jax
import jax.numpy as jnp
from jax.experimental import pallas as pl
from jax.experimental.pallas import tpu as pltpu

N_NODES = 50000
N_EDGES = 800000
NUM_GRAPHS = 128
FEAT = 64
HID = 128
NB = 400
N_NBLK = N_NODES // NB
EB = 3200
N_EBLK = N_EDGES // EB


def _build_conv_mat(conv_w, conv_b):
    r = np.arange(768)[:, None]
    i = r // 256
    yi = (r % 256) // 16
    xi = r % 16
    col = np.arange(2048)[None, :]
    q = col // 512
    rem = col % 512
    c = rem // 64
    y = (rem % 64) // 8
    x = rem % 8
    Y = 2 * y + q // 2
    X = 2 * x + q % 2
    ky = yi - Y + 1
    kx = xi - X + 1
    mask = (ky >= 0) & (ky < 3) & (kx >= 0) & (kx < 3)
    cB = np.broadcast_to(c, mask.shape)
    iB = np.broadcast_to(i, mask.shape)
    kyc = np.clip(ky, 0, 2)
    kxc = np.clip(kx, 0, 2)
    K = jnp.where(jnp.asarray(mask), conv_w[cB, iB, kyc, kxc], 0.0)
    bias = conv_b[c[0]]
    return K.astype(jnp.float32), bias.astype(jnp.float32)[None, :]


def _cnn_body(x_ref, K_ref, cb_ref, lw_ref, lb_ref, gw_ref, o_ref):
    h = jnp.dot(x_ref[...], K_ref[...], preferred_element_type=jnp.float32)
    h = jnp.maximum(h + cb_ref[...], 0.0)
    p = jnp.maximum(jnp.maximum(h[:, 0:512], h[:, 512:1024]),
                    jnp.maximum(h[:, 1024:1536], h[:, 1536:2048]))
    n = jnp.maximum(
        jnp.dot(p, lw_ref[...], preferred_element_type=jnp.float32)
        + lb_ref[...], 0.0)
    o_ref[...] = jnp.dot(n, gw_ref[...], preferred_element_type=jnp.float32)


def _deg_body(idx_ref, deg_ref):
    @pl.when(pl.program_id(0) == 0)
    def _init():
        deg_ref[...] = jnp.ones_like(deg_ref)

    def step(e, _):
        d = idx_ref[1, e]
        deg_ref[pl.ds(d, 1), :] += 1.0
        return _
    jax.lax.fori_loop(0, EB, step, None)


def _scale_body(xw_ref, deg_ref, y_ref):
    y_ref[...] = xw_ref[...] * jax.lax.rsqrt(deg_ref[...])


def _msg_body(idx_ref, y_ref, u_ref):
    @pl.when(pl.program_id(0) == 0)
    def _init():
        u_ref[...] = jnp.zeros_like(u_ref)

    def step(e, _):
        s = idx_ref[0, e]
        d = idx_ref[1, e]
        u_ref[pl.ds(d, 1), :] += y_ref[pl.ds(s, 1), :]
        return _
    jax.lax.fori_loop(0, EB, step, None)


def _pool_body(u_ref, xw_ref, deg_ref, bat_ref, gb_ref, m1w_ref, m1b_ref,
               m2w_ref, m2b_ref, o_ref, sums, counts):
    pid = pl.program_id(0)

    @pl.when(pid == 0)
    def _init():
        sums[...] = jnp.zeros_like(sums)
        counts[...] = jnp.zeros_like(counts)

    dinv = jax.lax.rsqrt(deg_ref[...])
    hg = jnp.maximum(dinv * u_ref[...] + dinv * dinv * xw_ref[...]
                     + gb_ref[...], 0.0)
    iota = jax.lax.broadcasted_iota(jnp.int32, (NB, NUM_GRAPHS), 1)
    onehot = (bat_ref[...] == iota).astype(jnp.float32)
    sums[...] += jax.lax.dot_general(
        onehot, hg, (((0,), (0,)), ((), ())),
        preferred_element_type=jnp.float32)
    counts[...] += jax.lax.dot_general(
        onehot, jnp.ones((NB, 1), jnp.float32), (((0,), (0,)), ((), ())),
        preferred_element_type=jnp.float32)

    @pl.when(pid == N_NBLK - 1)
    def _final():
        gmean = sums[...] / jnp.maximum(counts[...], 1.0)
        t = jnp.maximum(
            jnp.dot(gmean, m1w_ref[...], preferred_element_type=jnp.float32)
            + m1b_ref[...], 0.0)
        o_ref[...] = (jnp.dot(t, m2w_ref[...],
                              preferred_element_type=jnp.float32)
                      + m2b_ref[...])


def kernel(x, edge_index, batch, conv_w, conv_b, lin_w, lin_b, gcn_w, gcn_b,
           m1_w, m1_b, m2_w, m2_b):
    xf = x.reshape(N_NODES, 768)
    ei = edge_index.astype(jnp.int32)
    bat = batch.astype(jnp.int32).reshape(N_NODES, 1)
    K, cb = _build_conv_mat(conv_w, conv_b)
    lwT = lin_w.T
    gwT = gcn_w.T
    m1T = m1_w.T
    m2T = m2_w.T

    full = lambda shape: pl.BlockSpec(shape, lambda i: (0,) * len(shape))

    xw = pl.pallas_call(
        _cnn_body,
        grid=(N_NBLK,),
        in_specs=[
            pl.BlockSpec((NB, 768), lambda i: (i, 0)),
            full((768, 2048)), full((1, 2048)),
            full((512, FEAT)), full((1, FEAT)),
            full((FEAT, HID)),
        ],
        out_specs=pl.BlockSpec((NB, HID), lambda i: (i, 0)),
        out_shape=jax.ShapeDtypeStruct((N_NODES, HID), jnp.float32),
    )(xf, K, cb, lwT, lin_b[None, :], gwT)

    deg = pl.pallas_call(
        _deg_body,
        grid=(N_EBLK,),
        in_specs=[pl.BlockSpec((2, EB), lambda e: (0, e),
                               memory_space=pltpu.SMEM)],
        out_specs=pl.BlockSpec((N_NODES, 1), lambda e: (0, 0)),
        out_shape=jax.ShapeDtypeStruct((N_NODES, 1), jnp.float32),
    )(ei)

    y = pl.pallas_call(
        _scale_body,
        grid=(N_NBLK,),
        in_specs=[pl.BlockSpec((NB, HID), lambda i: (i, 0)),
                  pl.BlockSpec((NB, 1), lambda i: (i, 0))],
        out_specs=pl.BlockSpec((NB, HID), lambda i: (i, 0)),
        out_shape=jax.ShapeDtypeStruct((N_NODES, HID), jnp.float32),
    )(xw, deg)

    u = pl.pallas_call(
        _msg_body,
        grid=(N_EBLK,),
        in_specs=[
            pl.BlockSpec((2, EB), lambda e: (0, e), memory_space=pltpu.SMEM),
            full((N_NODES, HID)),
        ],
        out_specs=pl.BlockSpec((N_NODES, HID), lambda e: (0, 0)),
        out_shape=jax.ShapeDtypeStruct((N_NODES, HID), jnp.float32),
    )(ei, y)

    out = pl.pallas_call(
        _pool_body,
        grid=(N_NBLK,),
        in_specs=[
            pl.BlockSpec((NB, HID), lambda i: (i, 0)),
            pl.BlockSpec((NB, HID), lambda i: (i, 0)),
            pl.BlockSpec((NB, 1), lambda i: (i, 0)),
            pl.BlockSpec((NB, 1), lambda i: (i, 0)),
            full((1, HID)),
            full((HID, HID // 2)), full((1, HID // 2)),
            full((HID // 2, 2)), full((1, 2)),
        ],
        out_specs=pl.BlockSpec((NUM_GRAPHS, 2), lambda i: (0, 0)),
        out_shape=jax.ShapeDtypeStruct((NUM_GRAPHS, 2), jnp.float32),
        scratch_shapes=[pltpu.VMEM((NUM_GRAPHS, HID), jnp.float32),
                        pltpu.VMEM((NUM_GRAPHS, 1), jnp.float32)],
    )(u, xw, deg, bat, gcn_b[None, :], m1T, m1_b[None, :], m2T, m2_b[None, :])
    return out


# speedup vs baseline: 1.1745x; 1.1745x over previous
import functools
import numpy as np
import jax
import jax.numpy as jnp
from jax.experimental import pallas as pl
from jax.experimental.pallas import tpu as pltpu

N_NODES = 50000
N_EDGES = 800000
NUM_GRAPHS = 128
FEAT = 64
HID = 128
NB = 400
N_NBLK = N_NODES // NB
EB = 3200
N_EBLK = N_EDGES // EB
UNROLL = 16


def _build_conv_mat(conv_w, conv_b):
    r = np.arange(768)[:, None]
    i = r // 256
    yi = (r % 256) // 16
    xi = r % 16
    col = np.arange(2048)[None, :]
    q = col // 512
    rem = col % 512
    c = rem // 64
    y = (rem % 64) // 8
    x = rem % 8
    Y = 2 * y + q // 2
    X = 2 * x + q % 2
    ky = yi - Y + 1
    kx = xi - X + 1
    mask = (ky >= 0) & (ky < 3) & (kx >= 0) & (kx < 3)
    cB = np.broadcast_to(c, mask.shape)
    iB = np.broadcast_to(i, mask.shape)
    kyc = np.clip(ky, 0, 2)
    kxc = np.clip(kx, 0, 2)
    K = jnp.where(jnp.asarray(mask), conv_w[cB, iB, kyc, kxc], 0.0)
    bias = conv_b[c[0]]
    return K.astype(jnp.float32), bias.astype(jnp.float32)[None, :]


def _cnn_body(x_ref, K_ref, cb_ref, lw_ref, lb_ref, gw_ref, o_ref):
    h = jnp.dot(x_ref[...], K_ref[...], preferred_element_type=jnp.float32)
    h = jnp.maximum(h + cb_ref[...], 0.0)
    p = jnp.maximum(jnp.maximum(h[:, 0:512], h[:, 512:1024]),
                    jnp.maximum(h[:, 1024:1536], h[:, 1536:2048]))
    n = jnp.maximum(
        jnp.dot(p, lw_ref[...], preferred_element_type=jnp.float32)
        + lb_ref[...], 0.0)
    o_ref[...] = jnp.dot(n, gw_ref[...], preferred_element_type=jnp.float32)


def _deg_body(idx_ref, deg_ref):
    @pl.when(pl.program_id(0) == 0)
    def _init():
        deg_ref[...] = jnp.ones_like(deg_ref)

    def step(o, _):
        base = o * UNROLL
        for k in range(UNROLL):
            d = idx_ref[1, base + k]
            deg_ref[pl.ds(d, 1), :] += 1.0
        return _
    jax.lax.fori_loop(0, EB // UNROLL, step, None)


def _scale_body(xw_ref, deg_ref, y_ref):
    y_ref[...] = xw_ref[...] * jax.lax.rsqrt(deg_ref[...])


def _msg_body(idx_ref, y_ref, u_ref):
    @pl.when(pl.program_id(0) == 0)
    def _init():
        u_ref[...] = jnp.zeros_like(u_ref)

    def step(o, _):
        base = o * UNROLL
        for k in range(UNROLL):
            e = base + k
            s = idx_ref[0, e]
            d = idx_ref[1, e]
            u_ref[pl.ds(d, 1), :] += y_ref[pl.ds(s, 1), :]
        return _
    jax.lax.fori_loop(0, EB // UNROLL, step, None)


def _pool_body(u_ref, xw_ref, deg_ref, bat_ref, gb_ref, m1w_ref, m1b_ref,
               m2w_ref, m2b_ref, o_ref, sums, counts):
    pid = pl.program_id(0)

    @pl.when(pid == 0)
    def _init():
        sums[...] = jnp.zeros_like(sums)
        counts[...] = jnp.zeros_like(counts)

    dinv = jax.lax.rsqrt(deg_ref[...])
    hg = jnp.maximum(dinv * u_ref[...] + dinv * dinv * xw_ref[...]
                     + gb_ref[...], 0.0)
    iota = jax.lax.broadcasted_iota(jnp.int32, (NB, NUM_GRAPHS), 1)
    onehot = (bat_ref[...] == iota).astype(jnp.float32)
    sums[...] += jax.lax.dot_general(
        onehot, hg, (((0,), (0,)), ((), ())),
        preferred_element_type=jnp.float32)
    counts[...] += jax.lax.dot_general(
        onehot, jnp.ones((NB, 1), jnp.float32), (((0,), (0,)), ((), ())),
        preferred_element_type=jnp.float32)

    @pl.when(pid == N_NBLK - 1)
    def _final():
        gmean = sums[...] / jnp.maximum(counts[...], 1.0)
        t = jnp.maximum(
            jnp.dot(gmean, m1w_ref[...], preferred_element_type=jnp.float32)
            + m1b_ref[...], 0.0)
        o_ref[...] = (jnp.dot(t, m2w_ref[...],
                              preferred_element_type=jnp.float32)
                      + m2b_ref[...])


def kernel(x, edge_index, batch, conv_w, conv_b, lin_w, lin_b, gcn_w, gcn_b,
           m1_w, m1_b, m2_w, m2_b):
    xf = x.reshape(N_NODES, 768)
    ei = edge_index.astype(jnp.int32)
    bat = batch.astype(jnp.int32).reshape(N_NODES, 1)
    K, cb = _build_conv_mat(conv_w, conv_b)
    lwT = lin_w.T
    gwT = gcn_w.T
    m1T = m1_w.T
    m2T = m2_w.T

    full = lambda shape: pl.BlockSpec(shape, lambda i: (0,) * len(shape))

    xw = pl.pallas_call(
        _cnn_body,
        grid=(N_NBLK,),
        in_specs=[
            pl.BlockSpec((NB, 768), lambda i: (i, 0)),
            full((768, 2048)), full((1, 2048)),
            full((512, FEAT)), full((1, FEAT)),
            full((FEAT, HID)),
        ],
        out_specs=pl.BlockSpec((NB, HID), lambda i: (i, 0)),
        out_shape=jax.ShapeDtypeStruct((N_NODES, HID), jnp.float32),
    )(xf, K, cb, lwT, lin_b[None, :], gwT)

    deg = pl.pallas_call(
        _deg_body,
        grid=(N_EBLK,),
        in_specs=[pl.BlockSpec((2, EB), lambda e: (0, e),
                               memory_space=pltpu.SMEM)],
        out_specs=pl.BlockSpec((N_NODES, 1), lambda e: (0, 0)),
        out_shape=jax.ShapeDtypeStruct((N_NODES, 1), jnp.float32),
    )(ei)

    y = pl.pallas_call(
        _scale_body,
        grid=(N_NBLK,),
        in_specs=[pl.BlockSpec((NB, HID), lambda i: (i, 0)),
                  pl.BlockSpec((NB, 1), lambda i: (i, 0))],
        out_specs=pl.BlockSpec((NB, HID), lambda i: (i, 0)),
        out_shape=jax.ShapeDtypeStruct((N_NODES, HID), jnp.float32),
    )(xw, deg)

    u = pl.pallas_call(
        _msg_body,
        grid=(N_EBLK,),
        in_specs=[
            pl.BlockSpec((2, EB), lambda e: (0, e), memory_space=pltpu.SMEM),
            full((N_NODES, HID)),
        ],
        out_specs=pl.BlockSpec((N_NODES, HID), lambda e: (0, 0)),
        out_shape=jax.ShapeDtypeStruct((N_NODES, HID), jnp.float32),
    )(ei, y)

    out = pl.pallas_call(
        _pool_body,
        grid=(N_NBLK,),
        in_specs=[
            pl.BlockSpec((NB, HID), lambda i: (i, 0)),
            pl.BlockSpec((NB, HID), lambda i: (i, 0)),
            pl.BlockSpec((NB, 1), lambda i: (i, 0)),
            pl.BlockSpec((NB, 1), lambda i: (i, 0)),
            full((1, HID)),
            full((HID, HID // 2)), full((1, HID // 2)),
            full((HID // 2, 2)), full((1, 2)),
        ],
        out_specs=pl.BlockSpec((NUM_GRAPHS, 2), lambda i: (0, 0)),
        out_shape=jax.ShapeDtypeStruct((NUM_GRAPHS, 2), jnp.float32),
        scratch_shapes=[pltpu.VMEM((NUM_GRAPHS, HID), jnp.float32),
                        pltpu.VMEM((NUM_GRAPHS, 1), jnp.float32)],
    )(u, xw, deg, bat, gcn_b[None, :], m1T, m1_b[None, :], m2T, m2_b[None, :])
    return out
